# baseline (speedup 1.0000x reference)
import jax
import jax.numpy as jnp
from jax import lax
from jax.experimental import pallas as pl
from jax.experimental.pallas import tpu as pltpu
from jax.experimental.pallas import tpu_sc as plsc

_H = 512
_W = 512
_NPK = 256
_NC = 2
_NS = 16
_L = 16
_NSLAB = 4
_SLAB = _H * _W // _NSLAB
_CDUMP = _SLAB // 4
_CAND = _CDUMP + _L
_SEL = 288
_SDUMP = 304
_SELB = _SDUMP + _L
_MRG = _NSLAB * _SEL
_FB = 304
_FBB = _FB + _L
_BIG = 0x3FFFFFFF


def _nms_body(s_ref, x_ref):
    s = s_ref[0]
    zrow = jnp.zeros((1, _W), jnp.float32)
    zcol = jnp.zeros((_H, 1), jnp.float32)
    w = jnp.concatenate([zcol, s[:, :-1]], axis=1)
    e = jnp.concatenate([s[:, 1:], zcol], axis=1)
    nw = jnp.concatenate([zrow, w[:-1, :]], axis=0)
    n = jnp.concatenate([zrow, s[:-1, :]], axis=0)
    ne = jnp.concatenate([zrow, e[:-1, :]], axis=0)
    sw = jnp.concatenate([w[1:, :], zrow], axis=0)
    so = jnp.concatenate([s[1:, :], zrow], axis=0)
    se = jnp.concatenate([e[1:, :], zrow], axis=0)
    mask = ((s > nw) & (s > n) & (s > ne) & (s > w)
            & (s >= e) & (s >= sw) & (s >= so) & (s >= se))
    x_ref[0] = jnp.where(mask, s, 0.0)


def _lane():
    return lax.broadcasted_iota(jnp.int32, (_L,), 0)


def _gat(x, idx):
    return x.at[idx].get(mode="promise_in_bounds")


def _prefix(x):
    lane = _lane()
    y = x
    for k in (1, 2, 4, 8):
        sh = _gat(y, jnp.maximum(lane - k, 0))
        y = y + jnp.where(lane >= k, sh, 0)
    return y


def _bcast(x, l):
    return _gat(x, jnp.full((_L,), l, jnp.int32))


def _unrolled(nvec, unroll, body, carry0):
    assert nvec % unroll == 0
    def step(i, car):
        for u in range(unroll):
            car = body(i * unroll + u, car)
        return car
    return lax.fori_loop(0, nvec // unroll, step, carry0)


def _count_ge_splat(buf, nvec, t_splat, unroll):
    def cstep(i, acc):
        bits = plsc.bitcast(buf[pl.ds(i * _L, _L)], jnp.int32)
        return acc + (bits >= t_splat).astype(jnp.int32)
    acc = _unrolled(nvec, unroll, cstep, jnp.zeros((_L,), jnp.int32))
    return _bcast(_prefix(acc), _L - 1)


def _search256(buf, nvec, unroll):
    lo = jnp.full((_L,), 1, jnp.int32)
    hi = jnp.full((_L,), 0x40000000, jnp.int32)
    for _ in range(31):
        mid = (lo + hi) >> 1
        ge = _count_ge_splat(buf, nvec, mid, unroll) >= _NPK
        lo = jnp.where(ge, mid, lo)
        hi = jnp.where(ge, hi, mid)
    return lo


def _compact(dst_v, dst_i, base, dump, v, ix, m):
    inc = _prefix(m.astype(jnp.int32))
    posv = jnp.minimum(jnp.where(m, base + inc - 1, dump), dump)
    plsc.store_scatter(dst_v, [posv], v)
    plsc.store_scatter(dst_i, [posv], ix)
    return base + _bcast(inc, _L - 1)


def _compress_ge(src_v, src_i, nvec, t_splat, dst_v, dst_i, dump, unroll):
    def sstep(i, sp):
        v = src_v[pl.ds(i * _L, _L)]
        m = plsc.bitcast(v, jnp.int32) >= t_splat
        ix = src_i[pl.ds(i * _L, _L)]
        return _compact(dst_v, dst_i, sp, dump, v, ix, m)
    return _unrolled(nvec, unroll, sstep, jnp.zeros((_L,), jnp.int32))


def _topk_body(xr_ref, out_ref, slab, valb, idxb, selv, seli,
               mval, midx, fval, fidx, outb, shv, shi):
    c = lax.axis_index("c")
    s_id = lax.axis_index("s")
    b = c * 4 + s_id // _NSLAB
    q = s_id % _NSLAB
    lane = _lane()
    pltpu.sync_copy(xr_ref.at[b, q], slab)
    gbase = q * _SLAB

    def zstep(i, _):
        valb[pl.ds(i * _L, _L)] = jnp.zeros((_L,), jnp.float32)
        return 0
    _unrolled(_CAND // _L, 5, zstep, 0)

    def compress_step(i, base):
        v = slab[pl.ds(i * _L, _L)]
        m = v > 0.0
        ix = lane + (gbase + i * _L)
        return _compact(valb, idxb, base, _CDUMP, v, ix, m)

    _unrolled(_SLAB // _L, 8, compress_step, jnp.zeros((_L,), jnp.int32))

    t_loc = _search256(valb, _CDUMP // _L, 16)
    def istep(i, _):
        selv[pl.ds(i * _L, _L)] = jnp.full((_L,), -1.0, jnp.float32)
        seli[pl.ds(i * _L, _L)] = jnp.full((_L,), _BIG, jnp.int32)
        return 0
    lax.fori_loop(0, _SEL // _L, istep, 0)
    _compress_ge(valb, idxb, _CDUMP // _L, t_loc, selv, seli, _SDUMP, 8)
    pltpu.sync_copy(selv.at[pl.ds(0, _SEL)], shv.at[pl.ds(s_id * _SEL, _SEL)])
    pltpu.sync_copy(seli.at[pl.ds(0, _SEL)], shi.at[pl.ds(s_id * _SEL, _SEL)])
    plsc.subcore_barrier()

    @pl.when(q == 0)
    def _merge():
        pltpu.sync_copy(shv.at[pl.ds(s_id * _SEL, _MRG)], mval)
        pltpu.sync_copy(shi.at[pl.ds(s_id * _SEL, _MRG)], midx)
        nvm = _MRG // _L
        t_img = _search256(mval, nvm, 8)
        def fstep(i, _):
            fval[pl.ds(i * _L, _L)] = jnp.full((_L,), -1.0, jnp.float32)
            fidx[pl.ds(i * _L, _L)] = jnp.full((_L,), _BIG, jnp.int32)
            return 0
        lax.fori_loop(0, _FB // _L, fstep, 0)
        _compress_ge(mval, midx, nvm, t_img, fval, fidx, _FB, 8)

        for kb in range(_FB // _L):
            fv = fval[pl.ds(kb * _L, _L)]
            fx = fidx[pl.ds(kb * _L, _L)]

            def rstep(t, rk, fv=fv, fx=fx):
                ov = fval[pl.ds(t * _L, _L)]
                ox = fidx[pl.ds(t * _L, _L)]
                for l in range(_L):
                    vb = _bcast(ov, l)
                    xb = _bcast(ox, l)
                    beats = (vb > fv) | ((vb == fv) & (xb < fx))
                    rk = rk + beats.astype(jnp.int32)
                return rk

            rk = lax.fori_loop(0, _FB // _L, rstep,
                               jnp.zeros((_L,), jnp.int32))
            wm = rk < _NPK
            plsc.store_scatter(outb, [jnp.where(wm, 2 * rk, 2 * _NPK)],
                               fx >> 9)
            plsc.store_scatter(
                outb, [jnp.where(wm, 2 * rk + 1, 2 * _NPK + 1)], fx & 511)
        pltpu.sync_copy(outb.at[pl.ds(0, 2 * _NPK)], out_ref.at[b])


def _sc_topk(x3):
    xr = x3.reshape(8, _NSLAB, _SLAB)
    mesh = plsc.VectorSubcoreMesh(
        core_axis_name="c", subcore_axis_name="s",
        num_cores=_NC, num_subcores=_NS)
    graph2 = pl.kernel(
        _topk_body,
        out_type=jax.ShapeDtypeStruct((8, 2 * _NPK), jnp.int32),
        mesh=mesh,
        compiler_params=pltpu.CompilerParams(needs_layout_passes=False),
        scratch_types=[
            pltpu.VMEM((_SLAB,), jnp.float32),
            pltpu.VMEM((_CAND,), jnp.float32),
            pltpu.VMEM((_CAND,), jnp.int32),
            pltpu.VMEM((_SELB,), jnp.float32),
            pltpu.VMEM((_SELB,), jnp.int32),
            pltpu.VMEM((_MRG,), jnp.float32),
            pltpu.VMEM((_MRG,), jnp.int32),
            pltpu.VMEM((_FBB,), jnp.float32),
            pltpu.VMEM((_FBB,), jnp.int32),
            pltpu.VMEM((2 * _NPK + 2,), jnp.int32),
            pltpu.VMEM_SHARED((_NS * _SEL,), jnp.float32),
            pltpu.VMEM_SHARED((_NS * _SEL,), jnp.int32),
        ],
    )(xr)
    return graph2.reshape(8, _NPK, 2)


def kernel(feat):
    B = feat.shape[0]
    s = jax.nn.sigmoid(feat)
    x3 = pl.pallas_call(
        _nms_body,
        grid=(B,),
        in_specs=[pl.BlockSpec((1, _H, _W), lambda b: (b, 0, 0))],
        out_specs=pl.BlockSpec((1, _H, _W), lambda b: (b, 0, 0)),
        out_shape=jax.ShapeDtypeStruct((B, _H, _W), jnp.float32),
    )(s[:, 0, :, :])
    graph = _sc_topk(x3)
    return (x3[:, None, :, :], graph)


# speedup vs baseline: 1.0496x; 1.0496x over previous
import jax
import jax.numpy as jnp
from jax import lax
from jax.experimental import pallas as pl
from jax.experimental.pallas import tpu as pltpu
from jax.experimental.pallas import tpu_sc as plsc

_H = 512
_W = 512
_NPK = 256
_NC = 2
_NS = 16
_L = 16
_NSLAB = 4
_SLAB = _H * _W // _NSLAB
_CDUMP = _SLAB // 4
_CAND = _CDUMP + _L
_SEL = 288
_SDUMP = 304
_SELB = _SDUMP + _L
_MRG = _NSLAB * _SEL
_FB = 304
_FBB = _FB + _L
_BIG = 0x3FFFFFFF


def _nms_body(s_ref, x_ref):
    s = s_ref[0]
    zrow = jnp.zeros((1, _W), jnp.float32)
    zcol = jnp.zeros((_H, 1), jnp.float32)
    w = jnp.concatenate([zcol, s[:, :-1]], axis=1)
    e = jnp.concatenate([s[:, 1:], zcol], axis=1)
    nw = jnp.concatenate([zrow, w[:-1, :]], axis=0)
    n = jnp.concatenate([zrow, s[:-1, :]], axis=0)
    ne = jnp.concatenate([zrow, e[:-1, :]], axis=0)
    sw = jnp.concatenate([w[1:, :], zrow], axis=0)
    so = jnp.concatenate([s[1:, :], zrow], axis=0)
    se = jnp.concatenate([e[1:, :], zrow], axis=0)
    mask = ((s > nw) & (s > n) & (s > ne) & (s > w)
            & (s >= e) & (s >= sw) & (s >= so) & (s >= se))
    x_ref[0] = jnp.where(mask, s, 0.0)


def _lane():
    return lax.broadcasted_iota(jnp.int32, (_L,), 0)


def _gat(x, idx):
    return x.at[idx].get(mode="promise_in_bounds")


def _prefix(x):
    lane = _lane()
    y = x
    for k in (1, 2, 4, 8):
        sh = _gat(y, jnp.maximum(lane - k, 0))
        y = y + jnp.where(lane >= k, sh, 0)
    return y


def _bcast(x, l):
    return _gat(x, jnp.full((_L,), l, jnp.int32))


def _unrolled(nvec, unroll, body, carry0):
    assert nvec % unroll == 0
    def step(i, car):
        for u in range(unroll):
            car = body(i * unroll + u, car)
        return car
    return lax.fori_loop(0, nvec // unroll, step, carry0)


def _count_ge_splat(buf, nvec, t_splat, unroll):
    def cstep(i, acc):
        bits = plsc.bitcast(buf[pl.ds(i * _L, _L)], jnp.int32)
        return acc + (bits >= t_splat).astype(jnp.int32)
    acc = _unrolled(nvec, unroll, cstep, jnp.zeros((_L,), jnp.int32))
    return _bcast(_prefix(acc), _L - 1)


def _search256(buf, nvec, unroll):
    lo = jnp.full((_L,), 1, jnp.int32)
    hi = jnp.full((_L,), 0x40000000, jnp.int32)
    for _ in range(31):
        mid = (lo + hi) >> 1
        ge = _count_ge_splat(buf, nvec, mid, unroll) >= _NPK
        lo = jnp.where(ge, mid, lo)
        hi = jnp.where(ge, hi, mid)
    return lo


def _compact(dst_v, dst_i, base, dump, v, ix, m):
    inc = _prefix(m.astype(jnp.int32))
    posv = jnp.minimum(jnp.where(m, base + inc - 1, dump), dump)
    plsc.store_scatter(dst_v, [posv], v)
    plsc.store_scatter(dst_i, [posv], ix)
    return base + _bcast(inc, _L - 1)


def _compress_ge(src_v, src_i, nvec, t_splat, dst_v, dst_i, dump, unroll):
    def sstep(i, sp):
        v = src_v[pl.ds(i * _L, _L)]
        m = plsc.bitcast(v, jnp.int32) >= t_splat
        ix = src_i[pl.ds(i * _L, _L)]
        return _compact(dst_v, dst_i, sp, dump, v, ix, m)
    return _unrolled(nvec, unroll, sstep, jnp.zeros((_L,), jnp.int32))


def _topk_body(xr_ref, out_ref, slab, valb, idxb, selv, seli,
               mval, midx, fval, fidx, outb, shv, shi):
    c = lax.axis_index("c")
    s_id = lax.axis_index("s")
    b = c * 4 + s_id // _NSLAB
    q = s_id % _NSLAB
    lane = _lane()
    pltpu.sync_copy(xr_ref.at[b, pl.ds(q * (_H // _NSLAB), _H // _NSLAB)],
                    slab)
    gbase = q * _SLAB

    def zstep(i, _):
        valb[pl.ds(i * _L, _L)] = jnp.zeros((_L,), jnp.float32)
        return 0
    _unrolled(_CAND // _L, 5, zstep, 0)

    def compress_step(i, base):
        v = slab[i >> 5, pl.ds((i & 31) * _L, _L)]
        m = v > 0.0
        ix = lane + (gbase + i * _L)
        return _compact(valb, idxb, base, _CDUMP, v, ix, m)

    _unrolled(_SLAB // _L, 4, compress_step, jnp.zeros((_L,), jnp.int32))

    t_loc = _search256(valb, _CDUMP // _L, 8)
    def istep(i, _):
        selv[pl.ds(i * _L, _L)] = jnp.full((_L,), -1.0, jnp.float32)
        seli[pl.ds(i * _L, _L)] = jnp.full((_L,), _BIG, jnp.int32)
        return 0
    lax.fori_loop(0, _SEL // _L, istep, 0)
    _compress_ge(valb, idxb, _CDUMP // _L, t_loc, selv, seli, _SDUMP, 8)
    pltpu.sync_copy(selv.at[pl.ds(0, _SEL)], shv.at[pl.ds(s_id * _SEL, _SEL)])
    pltpu.sync_copy(seli.at[pl.ds(0, _SEL)], shi.at[pl.ds(s_id * _SEL, _SEL)])
    plsc.subcore_barrier()

    @pl.when(q == 0)
    def _merge():
        pltpu.sync_copy(shv.at[pl.ds(s_id * _SEL, _MRG)], mval)
        pltpu.sync_copy(shi.at[pl.ds(s_id * _SEL, _MRG)], midx)
        nvm = _MRG // _L
        t_img = _search256(mval, nvm, 8)
        def fstep(i, _):
            fval[pl.ds(i * _L, _L)] = jnp.full((_L,), -1.0, jnp.float32)
            fidx[pl.ds(i * _L, _L)] = jnp.full((_L,), _BIG, jnp.int32)
            return 0
        lax.fori_loop(0, _FB // _L, fstep, 0)
        _compress_ge(mval, midx, nvm, t_img, fval, fidx, _FB, 8)

        for kb in range(_FB // _L):
            fv = fval[pl.ds(kb * _L, _L)]
            fx = fidx[pl.ds(kb * _L, _L)]

            def rstep(t, rk, fv=fv, fx=fx):
                ov = fval[pl.ds(t * _L, _L)]
                ox = fidx[pl.ds(t * _L, _L)]
                for l in range(_L):
                    vb = _bcast(ov, l)
                    xb = _bcast(ox, l)
                    beats = (vb > fv) | ((vb == fv) & (xb < fx))
                    rk = rk + beats.astype(jnp.int32)
                return rk

            rk = lax.fori_loop(0, _FB // _L, rstep,
                               jnp.zeros((_L,), jnp.int32))
            wm = rk < _NPK
            plsc.store_scatter(outb, [jnp.where(wm, 2 * rk, 2 * _NPK)],
                               fx >> 9)
            plsc.store_scatter(
                outb, [jnp.where(wm, 2 * rk + 1, 2 * _NPK + 1)], fx & 511)
        pltpu.sync_copy(outb.at[pl.ds(0, 2 * _NPK)], out_ref.at[b])


def _sc_topk(x3):
    mesh = plsc.VectorSubcoreMesh(
        core_axis_name="c", subcore_axis_name="s",
        num_cores=_NC, num_subcores=_NS)
    graph2 = pl.kernel(
        _topk_body,
        out_type=jax.ShapeDtypeStruct((8, 2 * _NPK), jnp.int32),
        mesh=mesh,
        compiler_params=pltpu.CompilerParams(needs_layout_passes=False),
        scratch_types=[
            pltpu.VMEM((_H // _NSLAB, _W), jnp.float32),
            pltpu.VMEM((_CAND,), jnp.float32),
            pltpu.VMEM((_CAND,), jnp.int32),
            pltpu.VMEM((_SELB,), jnp.float32),
            pltpu.VMEM((_SELB,), jnp.int32),
            pltpu.VMEM((_MRG,), jnp.float32),
            pltpu.VMEM((_MRG,), jnp.int32),
            pltpu.VMEM((_FBB,), jnp.float32),
            pltpu.VMEM((_FBB,), jnp.int32),
            pltpu.VMEM((2 * _NPK + 2,), jnp.int32),
            pltpu.VMEM_SHARED((_NS * _SEL,), jnp.float32),
            pltpu.VMEM_SHARED((_NS * _SEL,), jnp.int32),
        ],
    )(x3)
    return graph2.reshape(8, _NPK, 2)


def kernel(feat):
    B = feat.shape[0]
    s = jax.nn.sigmoid(feat)
    x3 = pl.pallas_call(
        _nms_body,
        grid=(B,),
        in_specs=[pl.BlockSpec((1, _H, _W), lambda b: (b, 0, 0))],
        out_specs=pl.BlockSpec((1, _H, _W), lambda b: (b, 0, 0)),
        out_shape=jax.ShapeDtypeStruct((B, _H, _W), jnp.float32),
    )(s[:, 0, :, :])
    graph = _sc_topk(x3)
    return (x3[:, None, :, :], graph)


# speedup vs baseline: 1.1066x; 1.0543x over previous
import jax
import jax.numpy as jnp
from jax import lax
from jax.experimental import pallas as pl
from jax.experimental.pallas import tpu as pltpu
from jax.experimental.pallas import tpu_sc as plsc

_H = 512
_W = 512
_NPK = 256
_NC = 2
_NS = 16
_L = 16
_NSLAB = 4
_SLAB = _H * _W // _NSLAB
_CDUMP = _SLAB // 4
_CAND = _CDUMP + _L
_SEL = 288
_SDUMP = 304
_SELB = _SDUMP + _L
_MRG = _NSLAB * _SEL
_FB = 304
_FBB = _FB + _L
_BIG = 0x3FFFFFFF
_OUTB = 528


def _nms_body(s_ref, x_ref):
    s = s_ref[0]
    zrow = jnp.zeros((1, _W), jnp.float32)
    zcol = jnp.zeros((_H, 1), jnp.float32)
    w = jnp.concatenate([zcol, s[:, :-1]], axis=1)
    e = jnp.concatenate([s[:, 1:], zcol], axis=1)
    nw = jnp.concatenate([zrow, w[:-1, :]], axis=0)
    n = jnp.concatenate([zrow, s[:-1, :]], axis=0)
    ne = jnp.concatenate([zrow, e[:-1, :]], axis=0)
    sw = jnp.concatenate([w[1:, :], zrow], axis=0)
    so = jnp.concatenate([s[1:, :], zrow], axis=0)
    se = jnp.concatenate([e[1:, :], zrow], axis=0)
    mask = ((s > nw) & (s > n) & (s > ne) & (s > w)
            & (s >= e) & (s >= sw) & (s >= so) & (s >= se))
    x_ref[0] = jnp.where(mask, s, 0.0)


def _lane():
    return lax.broadcasted_iota(jnp.int32, (_L,), 0)


def _gat(x, idx):
    return x.at[idx].get(mode="promise_in_bounds")


def _prefix(x):
    lane = _lane()
    y = x
    for k in (1, 2, 4, 8):
        sh = _gat(y, jnp.maximum(lane - k, 0))
        y = y + jnp.where(lane >= k, sh, 0)
    return y


def _bcast(x, l):
    return _gat(x, jnp.full((_L,), l, jnp.int32))


def _unrolled(nvec, unroll, body, carry0):
    assert nvec % unroll == 0
    def step(i, car):
        for u in range(unroll):
            car = body(i * unroll + u, car)
        return car
    return lax.fori_loop(0, nvec // unroll, step, carry0)


def _count_ge_splat(buf, nvec, t_splat, unroll):
    def cstep(i, acc):
        bits = plsc.bitcast(buf[pl.ds(i * _L, _L)], jnp.int32)
        return acc + (bits >= t_splat).astype(jnp.int32)
    acc = _unrolled(nvec, unroll, cstep, jnp.zeros((_L,), jnp.int32))
    return _bcast(_prefix(acc), _L - 1)


def _search256(buf, nvec, unroll):
    lo = jnp.full((_L,), 1, jnp.int32)
    hi = jnp.full((_L,), 0x40000000, jnp.int32)
    for _ in range(31):
        mid = (lo + hi) >> 1
        ge = _count_ge_splat(buf, nvec, mid, unroll) >= _NPK
        lo = jnp.where(ge, mid, lo)
        hi = jnp.where(ge, hi, mid)
    return lo


def _compact(dst_v, dst_i, base, dump, v, ix, m):
    inc = _prefix(m.astype(jnp.int32))
    posv = jnp.minimum(jnp.where(m, base + inc - 1, dump), dump)
    plsc.store_scatter(dst_v, [posv], v)
    plsc.store_scatter(dst_i, [posv], ix)
    return base + _bcast(inc, _L - 1)


def _compress_ge(src_v, src_i, nvec, t_splat, dst_v, dst_i, dump, unroll):
    def sstep(i, sp):
        v = src_v[pl.ds(i * _L, _L)]
        m = plsc.bitcast(v, jnp.int32) >= t_splat
        ix = src_i[pl.ds(i * _L, _L)]
        return _compact(dst_v, dst_i, sp, dump, v, ix, m)
    return _unrolled(nvec, unroll, sstep, jnp.zeros((_L,), jnp.int32))


def _topk_body(xr_ref, out_ref, slab, valb, idxb, selv, seli,
               mval, midx, fval, fidx, outb, red, shv, sho, shi):
    c = lax.axis_index("c")
    s_id = lax.axis_index("s")
    b = c * 4 + s_id // _NSLAB
    q = s_id % _NSLAB
    lane = _lane()
    pltpu.sync_copy(xr_ref.at[b, pl.ds(q * (_H // _NSLAB), _H // _NSLAB)],
                    slab)
    gbase = q * _SLAB

    def zstep(i, _):
        valb[pl.ds(i * _L, _L)] = jnp.zeros((_L,), jnp.float32)
        return 0
    _unrolled(_CAND // _L, 5, zstep, 0)

    def compress_step(i, base):
        v = slab[i >> 5, pl.ds((i & 31) * _L, _L)]
        m = v > 0.0
        ix = lane + (gbase + i * _L)
        return _compact(valb, idxb, base, _CDUMP, v, ix, m)

    _unrolled(_SLAB // _L, 4, compress_step, jnp.zeros((_L,), jnp.int32))

    t_loc = _search256(valb, _CDUMP // _L, 8)
    def istep(i, _):
        selv[pl.ds(i * _L, _L)] = jnp.full((_L,), -1.0, jnp.float32)
        seli[pl.ds(i * _L, _L)] = jnp.full((_L,), _BIG, jnp.int32)
        return 0
    lax.fori_loop(0, _SEL // _L, istep, 0)
    _compress_ge(valb, idxb, _CDUMP // _L, t_loc, selv, seli, _SDUMP, 8)
    pltpu.sync_copy(selv.at[pl.ds(0, _SEL)], shv.at[pl.ds(s_id * _SEL, _SEL)])
    pltpu.sync_copy(seli.at[pl.ds(0, _SEL)], shi.at[pl.ds(s_id * _SEL, _SEL)])
    plsc.subcore_barrier()

    lead = s_id - q
    pltpu.sync_copy(shv.at[pl.ds(lead * _SEL, _MRG)], mval)
    pltpu.sync_copy(shi.at[pl.ds(lead * _SEL, _MRG)], midx)
    nvm = _MRG // _L
    t_img = _search256(mval, nvm, 8)
    def fstep(i, _):
        fval[pl.ds(i * _L, _L)] = jnp.full((_L,), -1.0, jnp.float32)
        fidx[pl.ds(i * _L, _L)] = jnp.full((_L,), _BIG, jnp.int32)
        return 0
    lax.fori_loop(0, _FB // _L, fstep, 0)
    _compress_ge(mval, midx, nvm, t_img, fval, fidx, _FB, 8)

    def ostep(i, _):
        outb[pl.ds(i * _L, _L)] = jnp.zeros((_L,), jnp.int32)
        return 0
    _unrolled(_OUTB // _L, 3, ostep, 0)

    for j in range(( _FB // _L + _NSLAB - 1) // _NSLAB):
        kb = q + _NSLAB * j

        @pl.when(kb < _FB // _L)
        def _rank(kb=kb):
            fv = fval[pl.ds(kb * _L, _L)]
            fx = fidx[pl.ds(kb * _L, _L)]

            def rstep(t, rk, fv=fv, fx=fx):
                ov = fval[pl.ds(t * _L, _L)]
                ox = fidx[pl.ds(t * _L, _L)]
                for l in range(_L):
                    vb = _bcast(ov, l)
                    xb = _bcast(ox, l)
                    beats = (vb > fv) | ((vb == fv) & (xb < fx))
                    rk = rk + beats.astype(jnp.int32)
                return rk

            rk = lax.fori_loop(0, _FB // _L, rstep,
                               jnp.zeros((_L,), jnp.int32))
            wm = rk < _NPK
            plsc.store_scatter(outb, [jnp.where(wm, 2 * rk, 2 * _NPK)],
                               fx >> 9)
            plsc.store_scatter(
                outb, [jnp.where(wm, 2 * rk + 1, 2 * _NPK + 1)], fx & 511)

    pltpu.sync_copy(outb, sho.at[pl.ds(s_id * _OUTB, _OUTB)])
    plsc.subcore_barrier()

    @pl.when(q == 0)
    def _emit():
        pltpu.sync_copy(sho.at[pl.ds(s_id * _OUTB, _NSLAB * _OUTB)], red)

        def addstep(i, _):
            o = (red[pl.ds(i * _L, _L)]
                 + red[pl.ds(_OUTB + i * _L, _L)]
                 + red[pl.ds(2 * _OUTB + i * _L, _L)]
                 + red[pl.ds(3 * _OUTB + i * _L, _L)])
            outb[pl.ds(i * _L, _L)] = o
            return 0
        _unrolled(2 * _NPK // _L, 4, addstep, 0)
        pltpu.sync_copy(outb.at[pl.ds(0, 2 * _NPK)], out_ref.at[b])


def _sc_topk(x3):
    mesh = plsc.VectorSubcoreMesh(
        core_axis_name="c", subcore_axis_name="s",
        num_cores=_NC, num_subcores=_NS)
    graph2 = pl.kernel(
        _topk_body,
        out_type=jax.ShapeDtypeStruct((8, 2 * _NPK), jnp.int32),
        mesh=mesh,
        compiler_params=pltpu.CompilerParams(needs_layout_passes=False),
        scratch_types=[
            pltpu.VMEM((_H // _NSLAB, _W), jnp.float32),
            pltpu.VMEM((_CAND,), jnp.float32),
            pltpu.VMEM((_CAND,), jnp.int32),
            pltpu.VMEM((_SELB,), jnp.float32),
            pltpu.VMEM((_SELB,), jnp.int32),
            pltpu.VMEM((_MRG,), jnp.float32),
            pltpu.VMEM((_MRG,), jnp.int32),
            pltpu.VMEM((_FBB,), jnp.float32),
            pltpu.VMEM((_FBB,), jnp.int32),
            pltpu.VMEM((_OUTB,), jnp.int32),
            pltpu.VMEM((_NSLAB * _OUTB,), jnp.int32),
            pltpu.VMEM_SHARED((_NS * _SEL,), jnp.float32),
            pltpu.VMEM_SHARED((_NS * _OUTB,), jnp.int32),
            pltpu.VMEM_SHARED((_NS * _SEL,), jnp.int32),
        ],
    )(x3)
    return graph2.reshape(8, _NPK, 2)


def kernel(feat):
    B = feat.shape[0]
    s = jax.nn.sigmoid(feat)
    x3 = pl.pallas_call(
        _nms_body,
        grid=(B,),
        in_specs=[pl.BlockSpec((1, _H, _W), lambda b: (b, 0, 0))],
        out_specs=pl.BlockSpec((1, _H, _W), lambda b: (b, 0, 0)),
        out_shape=jax.ShapeDtypeStruct((B, _H, _W), jnp.float32),
    )(s[:, 0, :, :])
    graph = _sc_topk(x3)
    return (x3[:, None, :, :], graph)
